# baseline (speedup 1.0000x reference)
import functools

import jax
import jax.numpy as jnp
from jax import lax
from jax.experimental import pallas as pl
from jax.experimental.pallas import tpu as pltpu
from jax.experimental.pallas import tpu_sc as plsc

N = 10000
E = 320000
D = 128

NPAD = 10240
ROWS_PER_TILE = 640
NW = 32
CH = 128
NCH = 80

EW = NCH * CH
EPAD = NW * EW
DUMMY = N

_mesh = plsc.VectorSubcoreMesh(core_axis_name="c", subcore_axis_name="s")


@functools.partial(
    pl.kernel,
    out_type=jax.ShapeDtypeStruct((2, NPAD), jnp.float32),
    mesh=_mesh,
    scratch_types=[
        pltpu.VMEM_SHARED((NPAD,), jnp.float32),
        pltpu.VMEM((NCH, CH), jnp.int32),
        pltpu.VMEM((CH,), jnp.float32),
        pltpu.VMEM((ROWS_PER_TILE,), jnp.float32),
    ],
)
def _deg_kernel(col_hbm, zeros_hbm, out_hbm, sdeg, colv, onesv, obuf):
    c = lax.axis_index("c")
    s = lax.axis_index("s")
    wid = c * 16 + s
    base = s * ROWS_PER_TILE
    pltpu.sync_copy(zeros_hbm, sdeg.at[pl.ds(base, ROWS_PER_TILE)])
    for l in range(CH // 16):
        onesv[pl.ds(l * 16, 16)] = jnp.ones((16,), jnp.float32)
    pltpu.sync_copy(col_hbm.at[wid], colv)
    plsc.subcore_barrier()

    def body(j, carry):
        pltpu.sync_copy(onesv, sdeg.at[colv.at[j]], add=True)
        return carry

    lax.fori_loop(0, NCH, body, 0)
    plsc.subcore_barrier()
    pltpu.sync_copy(sdeg.at[pl.ds(base, ROWS_PER_TILE)], obuf)
    pltpu.sync_copy(obuf, out_hbm.at[c, pl.ds(base, ROWS_PER_TILE)])


@functools.partial(
    pl.kernel,
    out_type=jax.ShapeDtypeStruct((2, NPAD, D), jnp.float32),
    mesh=_mesh,
    scratch_types=[
        pltpu.VMEM_SHARED((NPAD, D), jnp.float32),
        pltpu.VMEM((NCH, CH), jnp.int32),
        pltpu.VMEM((NCH, CH), jnp.int32),
        pltpu.VMEM((CH, D), jnp.float32),
        pltpu.SemaphoreType.DMA,
    ],
)
def _spmm_kernel(row_hbm, col_hbm, h_hbm, zeros_hbm, out_hbm,
                 sacc, rowv, colv, gbuf0, sem0):
    c = lax.axis_index("c")
    s = lax.axis_index("s")
    wid = c * 16 + s
    base = s * ROWS_PER_TILE
    pltpu.sync_copy(zeros_hbm, sacc.at[pl.ds(base, ROWS_PER_TILE)])
    pltpu.sync_copy(row_hbm.at[wid], rowv)
    pltpu.sync_copy(col_hbm.at[wid], colv)
    plsc.subcore_barrier()

    def body(j, carry):
        pltpu.async_copy(h_hbm.at[rowv.at[j]], gbuf0, sem0).wait()
        pltpu.sync_copy(gbuf0, sacc.at[colv.at[j]], add=True)
        return carry

    lax.fori_loop(0, NCH, body, 0)
    plsc.subcore_barrier()
    for k in range(ROWS_PER_TILE // 128):
        pltpu.sync_copy(sacc.at[pl.ds(base + k * 128, 128)], gbuf0)
        pltpu.sync_copy(gbuf0, out_hbm.at[c, pl.ds(base + k * 128, 128)])


_RB = 640


def _tc1_body(x_ref, w_ref, h0_ref, h1_ref, o_ref):
    deg = h0_ref[...] + h1_ref[...] + 1.0
    dis = lax.rsqrt(deg)
    o_ref[...] = jnp.dot(x_ref[...], w_ref[...],
                         preferred_element_type=jnp.float32) * dis


def _tc1(x, W1, h0, h1):
    return pl.pallas_call(
        _tc1_body,
        grid=(NPAD // _RB,),
        in_specs=[
            pl.BlockSpec((_RB, D), lambda i: (i, 0)),
            pl.BlockSpec((D, D), lambda i: (0, 0)),
            pl.BlockSpec((_RB, 1), lambda i: (i, 0)),
            pl.BlockSpec((_RB, 1), lambda i: (i, 0)),
        ],
        out_specs=pl.BlockSpec((_RB, D), lambda i: (i, 0)),
        out_shape=jax.ShapeDtypeStruct((NPAD, D), jnp.float32),
    )(x, W1, h0, h1)


def _tc2_body(a0_ref, a1_ref, hs_ref, h0_ref, h1_ref, w_ref, b_ref, o_ref):
    deg = h0_ref[...] + h1_ref[...] + 1.0
    dis = lax.rsqrt(deg)
    pre = dis * (a0_ref[...] + a1_ref[...] + hs_ref[...]) + b_ref[...]
    z1 = jnp.maximum(pre, 0.0)
    o_ref[...] = jnp.dot(z1, w_ref[...],
                         preferred_element_type=jnp.float32) * dis


def _tc2(a0, a1, hs, h0, h1, W2, b1):
    return pl.pallas_call(
        _tc2_body,
        grid=(NPAD // _RB,),
        in_specs=[
            pl.BlockSpec((_RB, D), lambda i: (i, 0)),
            pl.BlockSpec((_RB, D), lambda i: (i, 0)),
            pl.BlockSpec((_RB, D), lambda i: (i, 0)),
            pl.BlockSpec((_RB, 1), lambda i: (i, 0)),
            pl.BlockSpec((_RB, 1), lambda i: (i, 0)),
            pl.BlockSpec((D, D), lambda i: (0, 0)),
            pl.BlockSpec((1, D), lambda i: (0, 0)),
        ],
        out_specs=pl.BlockSpec((_RB, D), lambda i: (i, 0)),
        out_shape=jax.ShapeDtypeStruct((NPAD, D), jnp.float32),
    )(a0, a1, hs, h0, h1, W2, b1)


def _tc3_body(a0_ref, a1_ref, hs_ref, h0_ref, h1_ref, b_ref, o_ref):
    deg = h0_ref[...] + h1_ref[...] + 1.0
    dis = lax.rsqrt(deg)
    o_ref[...] = dis * (a0_ref[...] + a1_ref[...] + hs_ref[...]) + b_ref[...]


def _tc3(a0, a1, hs, h0, h1, b2):
    return pl.pallas_call(
        _tc3_body,
        grid=(NPAD // _RB,),
        in_specs=[
            pl.BlockSpec((_RB, D), lambda i: (i, 0)),
            pl.BlockSpec((_RB, D), lambda i: (i, 0)),
            pl.BlockSpec((_RB, D), lambda i: (i, 0)),
            pl.BlockSpec((_RB, 1), lambda i: (i, 0)),
            pl.BlockSpec((_RB, 1), lambda i: (i, 0)),
            pl.BlockSpec((1, D), lambda i: (0, 0)),
        ],
        out_specs=pl.BlockSpec((_RB, D), lambda i: (i, 0)),
        out_shape=jax.ShapeDtypeStruct((NPAD, D), jnp.float32),
    )(a0, a1, hs, h0, h1, b2)


@jax.jit
def kernel(x, edge_index, W1, b1, W2, b2):
    row = edge_index[0]
    col = edge_index[1]
    row_p = jnp.concatenate(
        [row, jnp.zeros((EPAD - E,), jnp.int32)]).reshape(NW, NCH, CH)
    col_p = jnp.concatenate(
        [col, jnp.full((EPAD - E,), DUMMY, jnp.int32)]).reshape(NW, NCH, CH)

    zeros1d = jnp.zeros((ROWS_PER_TILE,), jnp.float32)
    zeros2d = jnp.zeros((ROWS_PER_TILE, D), jnp.float32)

    hist = _deg_kernel(col_p, zeros1d)
    h0 = hist[0][:, None]
    h1 = hist[1][:, None]

    x_pad = jnp.concatenate(
        [x, jnp.zeros((NPAD - N, D), jnp.float32)], axis=0)

    h1s = _tc1(x_pad, W1, h0, h1)
    agg1 = _spmm_kernel(row_p, col_p, h1s, zeros2d)
    h2s = _tc2(agg1[0], agg1[1], h1s, h0, h1, W2, b1.reshape(1, D))
    agg2 = _spmm_kernel(row_p, col_p, h2s, zeros2d)
    z = _tc3(agg2[0], agg2[1], h2s, h0, h1, b2.reshape(1, D))
    return z[:N]


# speedup vs baseline: 1.0002x; 1.0002x over previous
import functools

import jax
import jax.numpy as jnp
from jax import lax
from jax.experimental import pallas as pl
from jax.experimental.pallas import tpu as pltpu
from jax.experimental.pallas import tpu_sc as plsc

N = 10000
E = 320000
D = 128

NPAD = 10240
ROWS_PER_TILE = 640
NW = 32
CH = 128
NCH = 80

EW = NCH * CH
EPAD = NW * EW
DUMMY = N

_mesh = plsc.VectorSubcoreMesh(core_axis_name="c", subcore_axis_name="s")


@functools.partial(
    pl.kernel,
    out_type=jax.ShapeDtypeStruct((2, NPAD), jnp.float32),
    mesh=_mesh,
    scratch_types=[
        pltpu.VMEM_SHARED((NPAD,), jnp.float32),
        pltpu.VMEM((NCH, CH), jnp.int32),
        pltpu.VMEM((CH,), jnp.float32),
        pltpu.VMEM((ROWS_PER_TILE,), jnp.float32),
    ],
)
def _deg_kernel(col_hbm, zeros_hbm, out_hbm, sdeg, colv, onesv, obuf):
    c = lax.axis_index("c")
    s = lax.axis_index("s")
    wid = c * 16 + s
    base = s * ROWS_PER_TILE
    pltpu.sync_copy(zeros_hbm, sdeg.at[pl.ds(base, ROWS_PER_TILE)])
    for l in range(CH // 16):
        onesv[pl.ds(l * 16, 16)] = jnp.ones((16,), jnp.float32)
    pltpu.sync_copy(col_hbm.at[wid], colv)
    plsc.subcore_barrier()

    def body(j, carry):
        pltpu.sync_copy(onesv, sdeg.at[colv.at[j]], add=True)
        return carry

    lax.fori_loop(0, NCH, body, 0)
    plsc.subcore_barrier()
    pltpu.sync_copy(sdeg.at[pl.ds(base, ROWS_PER_TILE)], obuf)
    pltpu.sync_copy(obuf, out_hbm.at[c, pl.ds(base, ROWS_PER_TILE)])


@functools.partial(
    pl.kernel,
    out_type=jax.ShapeDtypeStruct((2, NPAD, D), jnp.float32),
    mesh=_mesh,
    scratch_types=[
        pltpu.VMEM_SHARED((NPAD, D), jnp.float32),
        pltpu.VMEM((NCH, CH), jnp.int32),
        pltpu.VMEM((NCH, CH), jnp.int32),
        pltpu.VMEM((CH, D), jnp.float32),
        pltpu.SemaphoreType.DMA,
    ],
)
def _spmm_kernel(row_hbm, col_hbm, h_hbm, zeros_hbm, out_hbm,
                 sacc, rowv, colv, gbuf0, sem0):
    c = lax.axis_index("c")
    s = lax.axis_index("s")
    wid = c * 16 + s
    base = s * ROWS_PER_TILE
    pltpu.sync_copy(zeros_hbm, sacc.at[pl.ds(base, ROWS_PER_TILE)])
    pltpu.sync_copy(row_hbm.at[wid], rowv)
    pltpu.sync_copy(col_hbm.at[wid], colv)
    plsc.subcore_barrier()

    def body(j, carry):
        pltpu.async_copy(h_hbm.at[rowv.at[j]], gbuf0, sem0).wait()
        pltpu.sync_copy(gbuf0, sacc.at[colv.at[j]], add=True)
        return carry

    lax.fori_loop(0, NCH, body, 0)
    plsc.subcore_barrier()
    for k in range(ROWS_PER_TILE // 128):
        pltpu.sync_copy(sacc.at[pl.ds(base + k * 128, 128)], gbuf0)
        pltpu.sync_copy(gbuf0, out_hbm.at[c, pl.ds(base + k * 128, 128)])


_RB = 640


def _tc1_body(x_ref, w_ref, h0_ref, h1_ref, o_ref):
    deg = h0_ref[...] + h1_ref[...] + 1.0
    dis = lax.rsqrt(deg)
    o_ref[...] = jnp.dot(x_ref[...], w_ref[...],
                         preferred_element_type=jnp.float32) * dis


def _tc1(x, W1, h0, h1):
    return pl.pallas_call(
        _tc1_body,
        grid=(NPAD // _RB,),
        in_specs=[
            pl.BlockSpec((_RB, D), lambda i: (i, 0)),
            pl.BlockSpec((D, D), lambda i: (0, 0)),
            pl.BlockSpec((_RB, 1), lambda i: (i, 0)),
            pl.BlockSpec((_RB, 1), lambda i: (i, 0)),
        ],
        out_specs=pl.BlockSpec((_RB, D), lambda i: (i, 0)),
        out_shape=jax.ShapeDtypeStruct((NPAD, D), jnp.float32),
    )(x, W1, h0, h1)


def _tc2_body(a0_ref, a1_ref, hs_ref, h0_ref, h1_ref, w_ref, b_ref, o_ref):
    deg = h0_ref[...] + h1_ref[...] + 1.0
    dis = lax.rsqrt(deg)
    pre = dis * (a0_ref[...] + a1_ref[...] + hs_ref[...]) + b_ref[...]
    z1 = jnp.maximum(pre, 0.0)
    o_ref[...] = jnp.dot(z1, w_ref[...],
                         preferred_element_type=jnp.float32) * dis


def _tc2(a0, a1, hs, h0, h1, W2, b1):
    return pl.pallas_call(
        _tc2_body,
        grid=(NPAD // _RB,),
        in_specs=[
            pl.BlockSpec((_RB, D), lambda i: (i, 0)),
            pl.BlockSpec((_RB, D), lambda i: (i, 0)),
            pl.BlockSpec((_RB, D), lambda i: (i, 0)),
            pl.BlockSpec((_RB, 1), lambda i: (i, 0)),
            pl.BlockSpec((_RB, 1), lambda i: (i, 0)),
            pl.BlockSpec((D, D), lambda i: (0, 0)),
            pl.BlockSpec((1, D), lambda i: (0, 0)),
        ],
        out_specs=pl.BlockSpec((_RB, D), lambda i: (i, 0)),
        out_shape=jax.ShapeDtypeStruct((NPAD, D), jnp.float32),
    )(a0, a1, hs, h0, h1, W2, b1)


def _tc3_body(a0_ref, a1_ref, hs_ref, h0_ref, h1_ref, b_ref, o_ref):
    deg = h0_ref[...] + h1_ref[...] + 1.0
    dis = lax.rsqrt(deg)
    o_ref[...] = dis * (a0_ref[...] + a1_ref[...] + hs_ref[...]) + b_ref[...]


def _tc3(a0, a1, hs, h0, h1, b2):
    return pl.pallas_call(
        _tc3_body,
        grid=(NPAD // _RB,),
        in_specs=[
            pl.BlockSpec((_RB, D), lambda i: (i, 0)),
            pl.BlockSpec((_RB, D), lambda i: (i, 0)),
            pl.BlockSpec((_RB, D), lambda i: (i, 0)),
            pl.BlockSpec((_RB, 1), lambda i: (i, 0)),
            pl.BlockSpec((_RB, 1), lambda i: (i, 0)),
            pl.BlockSpec((1, D), lambda i: (0, 0)),
        ],
        out_specs=pl.BlockSpec((_RB, D), lambda i: (i, 0)),
        out_shape=jax.ShapeDtypeStruct((NPAD, D), jnp.float32),
    )(a0, a1, hs, h0, h1, b2)


@jax.jit
def kernel(x, edge_index, W1, b1, W2, b2):
    row = edge_index[0]
    col = edge_index[1]
    row_p = jnp.concatenate(
        [row, jnp.zeros((EPAD - E,), jnp.int32)]).reshape(NW, NCH, CH)
    pad_dst = DUMMY + (jnp.arange(EPAD - E, dtype=jnp.int32) % (NPAD - N))
    col_p = jnp.concatenate([col, pad_dst]).reshape(NW, NCH, CH)

    zeros1d = jnp.zeros((ROWS_PER_TILE,), jnp.float32)
    zeros2d = jnp.zeros((ROWS_PER_TILE, D), jnp.float32)

    hist = _deg_kernel(col_p, zeros1d)
    h0 = hist[0][:, None]
    h1 = hist[1][:, None]

    x_pad = jnp.concatenate(
        [x, jnp.zeros((NPAD - N, D), jnp.float32)], axis=0)

    h1s = _tc1(x_pad, W1, h0, h1)
    agg1 = _spmm_kernel(row_p, col_p, h1s, zeros2d)
    h2s = _tc2(agg1[0], agg1[1], h1s, h0, h1, W2, b1.reshape(1, D))
    agg2 = _spmm_kernel(row_p, col_p, h2s, zeros2d)
    z = _tc3(agg2[0], agg2[1], h2s, h0, h1, b2.reshape(1, D))
    return z[:N]


# speedup vs baseline: 1.5125x; 1.5123x over previous
import functools

import jax
import jax.numpy as jnp
from jax import lax
from jax.experimental import pallas as pl
from jax.experimental.pallas import tpu as pltpu
from jax.experimental.pallas import tpu_sc as plsc

N = 10000
E = 320000
D = 128

NPAD = 10240
ROWS_PER_TILE = 640
NW = 32
CH = 128
NCH = 79

EW = NCH * CH
EPAD = NW * EW
DUMMY = N

_mesh = plsc.VectorSubcoreMesh(core_axis_name="c", subcore_axis_name="s")


@functools.partial(
    pl.kernel,
    out_type=jax.ShapeDtypeStruct((2, NPAD), jnp.float32),
    mesh=_mesh,
    scratch_types=[
        pltpu.VMEM_SHARED((NPAD,), jnp.float32),
        pltpu.VMEM((NCH, CH), jnp.int32),
        pltpu.VMEM((CH,), jnp.float32),
        pltpu.VMEM((ROWS_PER_TILE,), jnp.float32),
    ],
)
def _deg_kernel(col_hbm, zeros_hbm, out_hbm, sdeg, colv, onesv, obuf):
    c = lax.axis_index("c")
    s = lax.axis_index("s")
    wid = c * 16 + s
    base = s * ROWS_PER_TILE
    pltpu.sync_copy(zeros_hbm, sdeg.at[pl.ds(base, ROWS_PER_TILE)])
    for l in range(CH // 16):
        onesv[pl.ds(l * 16, 16)] = jnp.ones((16,), jnp.float32)
    pltpu.sync_copy(col_hbm.at[wid], colv)
    plsc.subcore_barrier()

    def body(j, carry):
        pltpu.sync_copy(onesv, sdeg.at[colv.at[j]], add=True)
        return carry

    lax.fori_loop(0, NCH, body, 0)
    plsc.subcore_barrier()
    pltpu.sync_copy(sdeg.at[pl.ds(base, ROWS_PER_TILE)], obuf)
    pltpu.sync_copy(obuf, out_hbm.at[c, pl.ds(base, ROWS_PER_TILE)])


@functools.partial(
    pl.kernel,
    out_type=jax.ShapeDtypeStruct((2, NPAD, D), jnp.float32),
    mesh=_mesh,
    scratch_types=[
        pltpu.VMEM_SHARED((NPAD, D), jnp.float32),
        pltpu.VMEM((NCH, CH), jnp.int32),
        pltpu.VMEM((NCH, CH), jnp.int32),
        pltpu.VMEM((CH, D), jnp.float32),
        pltpu.SemaphoreType.DMA,
    ],
)
def _spmm_kernel(row_hbm, col_hbm, h_hbm, zeros_hbm, out_hbm,
                 sacc, rowv, colv, gbuf0, sem0):
    c = lax.axis_index("c")
    s = lax.axis_index("s")
    wid = c * 16 + s
    base = s * ROWS_PER_TILE
    pltpu.sync_copy(zeros_hbm, sacc.at[pl.ds(base, ROWS_PER_TILE)])
    pltpu.sync_copy(row_hbm.at[wid], rowv)
    pltpu.sync_copy(col_hbm.at[wid], colv)
    plsc.subcore_barrier()

    def body(j, carry):
        pltpu.async_copy(h_hbm.at[rowv.at[j]], gbuf0, sem0).wait()
        pltpu.sync_copy(gbuf0, sacc.at[colv.at[j]], add=True)
        return carry

    lax.fori_loop(0, NCH, body, 0)
    plsc.subcore_barrier()
    for k in range(ROWS_PER_TILE // 128):
        pltpu.sync_copy(sacc.at[pl.ds(base + k * 128, 128)], gbuf0)
        pltpu.sync_copy(gbuf0, out_hbm.at[c, pl.ds(base + k * 128, 128)])


_RB = 640


def _tc1_body(x_ref, w_ref, h0_ref, h1_ref, o_ref):
    deg = h0_ref[...] + h1_ref[...] + 1.0
    dis = lax.rsqrt(deg)
    o_ref[...] = jnp.dot(x_ref[...], w_ref[...],
                         preferred_element_type=jnp.float32) * dis


def _tc1(x, W1, h0, h1):
    return pl.pallas_call(
        _tc1_body,
        grid=(NPAD // _RB,),
        in_specs=[
            pl.BlockSpec((_RB, D), lambda i: (i, 0)),
            pl.BlockSpec((D, D), lambda i: (0, 0)),
            pl.BlockSpec((_RB, 1), lambda i: (i, 0)),
            pl.BlockSpec((_RB, 1), lambda i: (i, 0)),
        ],
        out_specs=pl.BlockSpec((_RB, D), lambda i: (i, 0)),
        out_shape=jax.ShapeDtypeStruct((NPAD, D), jnp.float32),
    )(x, W1, h0, h1)


def _tc2_body(a0_ref, a1_ref, hs_ref, h0_ref, h1_ref, w_ref, b_ref, o_ref):
    deg = h0_ref[...] + h1_ref[...] + 1.0
    dis = lax.rsqrt(deg)
    pre = dis * (a0_ref[...] + a1_ref[...] + hs_ref[...]) + b_ref[...]
    z1 = jnp.maximum(pre, 0.0)
    o_ref[...] = jnp.dot(z1, w_ref[...],
                         preferred_element_type=jnp.float32) * dis


def _tc2(a0, a1, hs, h0, h1, W2, b1):
    return pl.pallas_call(
        _tc2_body,
        grid=(NPAD // _RB,),
        in_specs=[
            pl.BlockSpec((_RB, D), lambda i: (i, 0)),
            pl.BlockSpec((_RB, D), lambda i: (i, 0)),
            pl.BlockSpec((_RB, D), lambda i: (i, 0)),
            pl.BlockSpec((_RB, 1), lambda i: (i, 0)),
            pl.BlockSpec((_RB, 1), lambda i: (i, 0)),
            pl.BlockSpec((D, D), lambda i: (0, 0)),
            pl.BlockSpec((1, D), lambda i: (0, 0)),
        ],
        out_specs=pl.BlockSpec((_RB, D), lambda i: (i, 0)),
        out_shape=jax.ShapeDtypeStruct((NPAD, D), jnp.float32),
    )(a0, a1, hs, h0, h1, W2, b1)


def _tc3_body(a0_ref, a1_ref, hs_ref, h0_ref, h1_ref, b_ref, o_ref):
    deg = h0_ref[...] + h1_ref[...] + 1.0
    dis = lax.rsqrt(deg)
    o_ref[...] = dis * (a0_ref[...] + a1_ref[...] + hs_ref[...]) + b_ref[...]


def _tc3(a0, a1, hs, h0, h1, b2):
    return pl.pallas_call(
        _tc3_body,
        grid=(NPAD // _RB,),
        in_specs=[
            pl.BlockSpec((_RB, D), lambda i: (i, 0)),
            pl.BlockSpec((_RB, D), lambda i: (i, 0)),
            pl.BlockSpec((_RB, D), lambda i: (i, 0)),
            pl.BlockSpec((_RB, 1), lambda i: (i, 0)),
            pl.BlockSpec((_RB, 1), lambda i: (i, 0)),
            pl.BlockSpec((1, D), lambda i: (0, 0)),
        ],
        out_specs=pl.BlockSpec((_RB, D), lambda i: (i, 0)),
        out_shape=jax.ShapeDtypeStruct((NPAD, D), jnp.float32),
    )(a0, a1, hs, h0, h1, b2)


@jax.jit
def kernel(x, edge_index, W1, b1, W2, b2):
    row = edge_index[0]
    col = edge_index[1]
    row_p = jnp.concatenate(
        [row, jnp.zeros((EPAD - E,), jnp.int32)]).reshape(NW, NCH, CH)
    pad_dst = DUMMY + (jnp.arange(EPAD - E, dtype=jnp.int32) % (NPAD - N))
    col_p = jnp.concatenate([col, pad_dst]).reshape(NW, NCH, CH)

    zeros1d = jnp.zeros((ROWS_PER_TILE,), jnp.float32)
    zeros2d = jnp.zeros((ROWS_PER_TILE, D), jnp.float32)

    hist = _deg_kernel(col_p, zeros1d)
    h0 = hist[0][:, None]
    h1 = hist[1][:, None]

    x_pad = jnp.concatenate(
        [x, jnp.zeros((NPAD - N, D), jnp.float32)], axis=0)

    h1s = _tc1(x_pad, W1, h0, h1)
    agg1 = _spmm_kernel(row_p, col_p, h1s, zeros2d)
    h2s = _tc2(agg1[0], agg1[1], h1s, h0, h1, W2, b1.reshape(1, D))
    agg2 = _spmm_kernel(row_p, col_p, h2s, zeros2d)
    z = _tc3(agg2[0], agg2[1], h2s, h0, h1, b2.reshape(1, D))
    return z[:N]


# speedup vs baseline: 1.7749x; 1.1735x over previous
import functools

import jax
import jax.numpy as jnp
from jax import lax
from jax.experimental import pallas as pl
from jax.experimental.pallas import tpu as pltpu
from jax.experimental.pallas import tpu_sc as plsc

N = 10000
E = 320000
D = 128

NPAD = 10240
ROWS_PER_TILE = 640
NW = 32
CH = 128
NCH = 79

EW = NCH * CH
EPAD = NW * EW
DUMMY = N

_mesh = plsc.VectorSubcoreMesh(core_axis_name="c", subcore_axis_name="s")


@functools.partial(
    pl.kernel,
    out_type=jax.ShapeDtypeStruct((2, NPAD), jnp.float32),
    mesh=_mesh,
    scratch_types=[
        pltpu.VMEM_SHARED((NPAD,), jnp.float32),
        pltpu.VMEM((NCH, CH), jnp.int32),
        pltpu.VMEM((CH,), jnp.float32),
        pltpu.VMEM((ROWS_PER_TILE,), jnp.float32),
    ],
)
def _deg_kernel(col_hbm, zeros_hbm, out_hbm, sdeg, colv, onesv, obuf):
    c = lax.axis_index("c")
    s = lax.axis_index("s")
    wid = c * 16 + s
    base = s * ROWS_PER_TILE
    pltpu.sync_copy(zeros_hbm, sdeg.at[pl.ds(base, ROWS_PER_TILE)])
    for l in range(CH // 16):
        onesv[pl.ds(l * 16, 16)] = jnp.ones((16,), jnp.float32)
    pltpu.sync_copy(col_hbm.at[wid], colv)
    plsc.subcore_barrier()

    def body(j, carry):
        pltpu.sync_copy(onesv, sdeg.at[colv.at[j]], add=True)
        return carry

    lax.fori_loop(0, NCH, body, 0)
    plsc.subcore_barrier()
    pltpu.sync_copy(sdeg.at[pl.ds(base, ROWS_PER_TILE)], obuf)
    pltpu.sync_copy(obuf, out_hbm.at[c, pl.ds(base, ROWS_PER_TILE)])


@functools.partial(
    pl.kernel,
    out_type=jax.ShapeDtypeStruct((2, NPAD, D), jnp.float32),
    mesh=_mesh,
    scratch_types=[
        pltpu.VMEM_SHARED((NPAD, D), jnp.float32),
        pltpu.VMEM((1, CH), jnp.int32),
        pltpu.VMEM((1, CH), jnp.int32),
        pltpu.VMEM((NCH, CH), jnp.int32),
        pltpu.VMEM((CH, D), jnp.float32),
        pltpu.VMEM((CH, D), jnp.float32),
        pltpu.SemaphoreType.DMA,
        pltpu.SemaphoreType.DMA,
        pltpu.SemaphoreType.DMA,
        pltpu.SemaphoreType.DMA,
    ],
)
def _spmm_kernel(row_hbm, col_hbm, h_hbm, zeros_hbm, out_hbm,
                 sacc, ri0, ri1, colv, gbuf0, gbuf1, gsem0, gsem1,
                 isem0, isem1):
    c = lax.axis_index("c")
    s = lax.axis_index("s")
    wid = c * 16 + s
    base = s * ROWS_PER_TILE
    pltpu.sync_copy(zeros_hbm, sacc.at[pl.ds(base, ROWS_PER_TILE)])
    pltpu.sync_copy(col_hbm.at[wid], colv)
    pltpu.sync_copy(row_hbm.at[wid, pl.ds(0, 1)], ri0)
    pltpu.sync_copy(row_hbm.at[wid, pl.ds(1, 1)], ri1)
    plsc.subcore_barrier()

    pltpu.async_copy(h_hbm.at[ri0.at[0]], gbuf0, gsem0)

    def body(i, carry):
        a = 2 * i
        b = a + 1
        pltpu.async_copy(h_hbm.at[ri1.at[0]], gbuf1, gsem1)
        pltpu.make_async_copy(h_hbm.at[ri0.at[0]], gbuf0, gsem0).wait()
        nxt0 = jnp.minimum(a + 2, NCH - 1)
        pltpu.async_copy(row_hbm.at[wid, pl.ds(nxt0, 1)], ri0, isem0)
        pltpu.sync_copy(gbuf0, sacc.at[colv.at[a]], add=True)
        pltpu.make_async_copy(row_hbm.at[wid, pl.ds(nxt0, 1)], ri0, isem0).wait()
        pltpu.async_copy(h_hbm.at[ri0.at[0]], gbuf0, gsem0)
        pltpu.make_async_copy(h_hbm.at[ri1.at[0]], gbuf1, gsem1).wait()
        nxt1 = jnp.minimum(b + 2, NCH - 1)
        pltpu.async_copy(row_hbm.at[wid, pl.ds(nxt1, 1)], ri1, isem1)
        pltpu.sync_copy(gbuf1, sacc.at[colv.at[b]], add=True)
        pltpu.make_async_copy(row_hbm.at[wid, pl.ds(nxt1, 1)], ri1, isem1).wait()
        return carry

    lax.fori_loop(0, NCH // 2, body, 0)
    pltpu.make_async_copy(h_hbm.at[ri0.at[0]], gbuf0, gsem0).wait()
    pltpu.sync_copy(gbuf0, sacc.at[colv.at[NCH - 1]], add=True)
    plsc.subcore_barrier()
    for k in range(ROWS_PER_TILE // 128):
        pltpu.sync_copy(sacc.at[pl.ds(base + k * 128, 128)], gbuf0)
        pltpu.sync_copy(gbuf0, out_hbm.at[c, pl.ds(base + k * 128, 128)])


_RB = 640


def _tc1_body(x_ref, w_ref, h0_ref, h1_ref, o_ref):
    deg = h0_ref[...] + h1_ref[...] + 1.0
    dis = lax.rsqrt(deg)
    o_ref[...] = jnp.dot(x_ref[...], w_ref[...],
                         preferred_element_type=jnp.float32) * dis


def _tc1(x, W1, h0, h1):
    return pl.pallas_call(
        _tc1_body,
        grid=(NPAD // _RB,),
        in_specs=[
            pl.BlockSpec((_RB, D), lambda i: (i, 0)),
            pl.BlockSpec((D, D), lambda i: (0, 0)),
            pl.BlockSpec((_RB, 1), lambda i: (i, 0)),
            pl.BlockSpec((_RB, 1), lambda i: (i, 0)),
        ],
        out_specs=pl.BlockSpec((_RB, D), lambda i: (i, 0)),
        out_shape=jax.ShapeDtypeStruct((NPAD, D), jnp.float32),
    )(x, W1, h0, h1)


def _tc2_body(a0_ref, a1_ref, hs_ref, h0_ref, h1_ref, w_ref, b_ref, o_ref):
    deg = h0_ref[...] + h1_ref[...] + 1.0
    dis = lax.rsqrt(deg)
    pre = dis * (a0_ref[...] + a1_ref[...] + hs_ref[...]) + b_ref[...]
    z1 = jnp.maximum(pre, 0.0)
    o_ref[...] = jnp.dot(z1, w_ref[...],
                         preferred_element_type=jnp.float32) * dis


def _tc2(a0, a1, hs, h0, h1, W2, b1):
    return pl.pallas_call(
        _tc2_body,
        grid=(NPAD // _RB,),
        in_specs=[
            pl.BlockSpec((_RB, D), lambda i: (i, 0)),
            pl.BlockSpec((_RB, D), lambda i: (i, 0)),
            pl.BlockSpec((_RB, D), lambda i: (i, 0)),
            pl.BlockSpec((_RB, 1), lambda i: (i, 0)),
            pl.BlockSpec((_RB, 1), lambda i: (i, 0)),
            pl.BlockSpec((D, D), lambda i: (0, 0)),
            pl.BlockSpec((1, D), lambda i: (0, 0)),
        ],
        out_specs=pl.BlockSpec((_RB, D), lambda i: (i, 0)),
        out_shape=jax.ShapeDtypeStruct((NPAD, D), jnp.float32),
    )(a0, a1, hs, h0, h1, W2, b1)


def _tc3_body(a0_ref, a1_ref, hs_ref, h0_ref, h1_ref, b_ref, o_ref):
    deg = h0_ref[...] + h1_ref[...] + 1.0
    dis = lax.rsqrt(deg)
    o_ref[...] = dis * (a0_ref[...] + a1_ref[...] + hs_ref[...]) + b_ref[...]


def _tc3(a0, a1, hs, h0, h1, b2):
    return pl.pallas_call(
        _tc3_body,
        grid=(NPAD // _RB,),
        in_specs=[
            pl.BlockSpec((_RB, D), lambda i: (i, 0)),
            pl.BlockSpec((_RB, D), lambda i: (i, 0)),
            pl.BlockSpec((_RB, D), lambda i: (i, 0)),
            pl.BlockSpec((_RB, 1), lambda i: (i, 0)),
            pl.BlockSpec((_RB, 1), lambda i: (i, 0)),
            pl.BlockSpec((1, D), lambda i: (0, 0)),
        ],
        out_specs=pl.BlockSpec((_RB, D), lambda i: (i, 0)),
        out_shape=jax.ShapeDtypeStruct((NPAD, D), jnp.float32),
    )(a0, a1, hs, h0, h1, b2)


@jax.jit
def kernel(x, edge_index, W1, b1, W2, b2):
    row = edge_index[0]
    col = edge_index[1]
    row_p = jnp.concatenate(
        [row, jnp.zeros((EPAD - E,), jnp.int32)]).reshape(NW, NCH, CH)
    pad_dst = DUMMY + (jnp.arange(EPAD - E, dtype=jnp.int32) % (NPAD - N))
    col_p = jnp.concatenate([col, pad_dst]).reshape(NW, NCH, CH)

    zeros1d = jnp.zeros((ROWS_PER_TILE,), jnp.float32)
    zeros2d = jnp.zeros((ROWS_PER_TILE, D), jnp.float32)

    hist = _deg_kernel(col_p, zeros1d)
    h0 = hist[0][:, None]
    h1 = hist[1][:, None]

    x_pad = jnp.concatenate(
        [x, jnp.zeros((NPAD - N, D), jnp.float32)], axis=0)

    h1s = _tc1(x_pad, W1, h0, h1)
    agg1 = _spmm_kernel(row_p, col_p, h1s, zeros2d)
    h2s = _tc2(agg1[0], agg1[1], h1s, h0, h1, W2, b1.reshape(1, D))
    agg2 = _spmm_kernel(row_p, col_p, h2s, zeros2d)
    z = _tc3(agg2[0], agg2[1], h2s, h0, h1, b2.reshape(1, D))
    return z[:N]


# speedup vs baseline: 1.9192x; 1.0813x over previous
import functools

import jax
import jax.numpy as jnp
from jax import lax
from jax.experimental import pallas as pl
from jax.experimental.pallas import tpu as pltpu
from jax.experimental.pallas import tpu_sc as plsc

N = 10000
E = 320000
D = 128

NPAD = 10240
ROWS_PER_TILE = 640
NW = 32
CH = 128
NCH = 79

EW = NCH * CH
EPAD = NW * EW
DUMMY = N

_mesh = plsc.VectorSubcoreMesh(core_axis_name="c", subcore_axis_name="s")


@functools.partial(
    pl.kernel,
    out_type=jax.ShapeDtypeStruct((2, NPAD), jnp.float32),
    mesh=_mesh,
    scratch_types=[
        pltpu.VMEM_SHARED((NPAD,), jnp.float32),
        pltpu.VMEM((NCH, CH), jnp.int32),
        pltpu.VMEM((CH,), jnp.float32),
        pltpu.VMEM((ROWS_PER_TILE,), jnp.float32),
    ],
)
def _deg_kernel(col_hbm, zeros_hbm, out_hbm, sdeg, colv, onesv, obuf):
    c = lax.axis_index("c")
    s = lax.axis_index("s")
    wid = c * 16 + s
    base = s * ROWS_PER_TILE
    pltpu.sync_copy(zeros_hbm, sdeg.at[pl.ds(base, ROWS_PER_TILE)])
    for l in range(CH // 16):
        onesv[pl.ds(l * 16, 16)] = jnp.ones((16,), jnp.float32)
    pltpu.sync_copy(col_hbm.at[wid], colv)
    plsc.subcore_barrier()

    def body(j, carry):
        pltpu.sync_copy(onesv, sdeg.at[colv.at[j]], add=True)
        return carry

    lax.fori_loop(0, NCH, body, 0)
    plsc.subcore_barrier()
    pltpu.sync_copy(sdeg.at[pl.ds(base, ROWS_PER_TILE)], obuf)
    pltpu.sync_copy(obuf, out_hbm.at[c, pl.ds(base, ROWS_PER_TILE)])


@functools.partial(
    pl.kernel,
    out_type=jax.ShapeDtypeStruct((2, NPAD, D), jnp.float32),
    mesh=_mesh,
    scratch_types=[
        pltpu.VMEM_SHARED((NPAD, D), jnp.float32),
        pltpu.VMEM((1, CH), jnp.int32),
        pltpu.VMEM((1, CH), jnp.int32),
        pltpu.VMEM((NCH, CH), jnp.int32),
        pltpu.VMEM((CH, D), jnp.float32),
        pltpu.VMEM((CH, D), jnp.float32),
        pltpu.SemaphoreType.DMA,
        pltpu.SemaphoreType.DMA,
        pltpu.SemaphoreType.DMA,
        pltpu.SemaphoreType.DMA,
    ],
)
def _spmm_kernel(row_hbm, col_hbm, h_hbm, zeros_hbm, out_hbm,
                 sacc, ri0, ri1, colv, gbuf0, gbuf1, gsem0, gsem1,
                 isem0, isem1):
    c = lax.axis_index("c")
    s = lax.axis_index("s")
    wid = c * 16 + s
    base = s * ROWS_PER_TILE
    hsrc = h_hbm.at[c]
    pltpu.sync_copy(zeros_hbm, sacc.at[pl.ds(base, ROWS_PER_TILE)])
    pltpu.sync_copy(col_hbm.at[wid], colv)
    pltpu.sync_copy(row_hbm.at[wid, pl.ds(0, 1)], ri0)
    pltpu.sync_copy(row_hbm.at[wid, pl.ds(1, 1)], ri1)
    plsc.subcore_barrier()

    pltpu.async_copy(hsrc.at[ri0.at[0]], gbuf0, gsem0)

    def body(i, carry):
        a = 2 * i
        b = a + 1
        pltpu.async_copy(hsrc.at[ri1.at[0]], gbuf1, gsem1)
        pltpu.make_async_copy(hsrc.at[ri0.at[0]], gbuf0, gsem0).wait()
        nxt0 = jnp.minimum(a + 2, NCH - 1)
        pltpu.async_copy(row_hbm.at[wid, pl.ds(nxt0, 1)], ri0, isem0)
        pltpu.sync_copy(gbuf0, sacc.at[colv.at[a]], add=True)
        pltpu.make_async_copy(row_hbm.at[wid, pl.ds(nxt0, 1)], ri0, isem0).wait()
        pltpu.async_copy(hsrc.at[ri0.at[0]], gbuf0, gsem0)
        pltpu.make_async_copy(hsrc.at[ri1.at[0]], gbuf1, gsem1).wait()
        nxt1 = jnp.minimum(b + 2, NCH - 1)
        pltpu.async_copy(row_hbm.at[wid, pl.ds(nxt1, 1)], ri1, isem1)
        pltpu.sync_copy(gbuf1, sacc.at[colv.at[b]], add=True)
        pltpu.make_async_copy(row_hbm.at[wid, pl.ds(nxt1, 1)], ri1, isem1).wait()
        return carry

    lax.fori_loop(0, NCH // 2, body, 0)
    pltpu.make_async_copy(hsrc.at[ri0.at[0]], gbuf0, gsem0).wait()
    pltpu.sync_copy(gbuf0, sacc.at[colv.at[NCH - 1]], add=True)
    plsc.subcore_barrier()
    for k in range(ROWS_PER_TILE // 128):
        pltpu.sync_copy(sacc.at[pl.ds(base + k * 128, 128)], gbuf0)
        pltpu.sync_copy(gbuf0, out_hbm.at[c, pl.ds(base + k * 128, 128)])


_RB = 640


def _tc1_body(x_ref, w_ref, h0_ref, h1_ref, o_ref, o2_ref):
    deg = h0_ref[...] + h1_ref[...] + 1.0
    dis = lax.rsqrt(deg)
    hs = jnp.dot(x_ref[...], w_ref[...],
                 preferred_element_type=jnp.float32) * dis
    o_ref[...] = hs
    o2_ref[...] = hs


def _tc1(x, W1, h0, h1):
    return pl.pallas_call(
        _tc1_body,
        grid=(NPAD // _RB,),
        in_specs=[
            pl.BlockSpec((_RB, D), lambda i: (i, 0)),
            pl.BlockSpec((D, D), lambda i: (0, 0)),
            pl.BlockSpec((_RB, 1), lambda i: (i, 0)),
            pl.BlockSpec((_RB, 1), lambda i: (i, 0)),
        ],
        out_specs=[pl.BlockSpec((_RB, D), lambda i: (i, 0)),
                   pl.BlockSpec((_RB, D), lambda i: (i, 0))],
        out_shape=[jax.ShapeDtypeStruct((NPAD, D), jnp.float32),
                   jax.ShapeDtypeStruct((NPAD, D), jnp.float32)],
    )(x, W1, h0, h1)


def _tc2_body(a0_ref, a1_ref, hs_ref, h0_ref, h1_ref, w_ref, b_ref, o_ref, o2_ref):
    deg = h0_ref[...] + h1_ref[...] + 1.0
    dis = lax.rsqrt(deg)
    pre = dis * (a0_ref[...] + a1_ref[...] + hs_ref[...]) + b_ref[...]
    z1 = jnp.maximum(pre, 0.0)
    h2s = jnp.dot(z1, w_ref[...],
                  preferred_element_type=jnp.float32) * dis
    o_ref[...] = h2s
    o2_ref[...] = h2s


def _tc2(a0, a1, hs, h0, h1, W2, b1):
    return pl.pallas_call(
        _tc2_body,
        grid=(NPAD // _RB,),
        in_specs=[
            pl.BlockSpec((_RB, D), lambda i: (i, 0)),
            pl.BlockSpec((_RB, D), lambda i: (i, 0)),
            pl.BlockSpec((_RB, D), lambda i: (i, 0)),
            pl.BlockSpec((_RB, 1), lambda i: (i, 0)),
            pl.BlockSpec((_RB, 1), lambda i: (i, 0)),
            pl.BlockSpec((D, D), lambda i: (0, 0)),
            pl.BlockSpec((1, D), lambda i: (0, 0)),
        ],
        out_specs=[pl.BlockSpec((_RB, D), lambda i: (i, 0)),
                   pl.BlockSpec((_RB, D), lambda i: (i, 0))],
        out_shape=[jax.ShapeDtypeStruct((NPAD, D), jnp.float32),
                   jax.ShapeDtypeStruct((NPAD, D), jnp.float32)],
    )(a0, a1, hs, h0, h1, W2, b1)


def _tc3_body(a0_ref, a1_ref, hs_ref, h0_ref, h1_ref, b_ref, o_ref):
    deg = h0_ref[...] + h1_ref[...] + 1.0
    dis = lax.rsqrt(deg)
    o_ref[...] = dis * (a0_ref[...] + a1_ref[...] + hs_ref[...]) + b_ref[...]


def _tc3(a0, a1, hs, h0, h1, b2):
    return pl.pallas_call(
        _tc3_body,
        grid=(NPAD // _RB,),
        in_specs=[
            pl.BlockSpec((_RB, D), lambda i: (i, 0)),
            pl.BlockSpec((_RB, D), lambda i: (i, 0)),
            pl.BlockSpec((_RB, D), lambda i: (i, 0)),
            pl.BlockSpec((_RB, 1), lambda i: (i, 0)),
            pl.BlockSpec((_RB, 1), lambda i: (i, 0)),
            pl.BlockSpec((1, D), lambda i: (0, 0)),
        ],
        out_specs=pl.BlockSpec((_RB, D), lambda i: (i, 0)),
        out_shape=jax.ShapeDtypeStruct((NPAD, D), jnp.float32),
    )(a0, a1, hs, h0, h1, b2)


@jax.jit
def kernel(x, edge_index, W1, b1, W2, b2):
    row = edge_index[0]
    col = edge_index[1]
    row_p = jnp.concatenate(
        [row, jnp.zeros((EPAD - E,), jnp.int32)]).reshape(NW, NCH, CH)
    pad_dst = DUMMY + (jnp.arange(EPAD - E, dtype=jnp.int32) % (NPAD - N))
    col_p = jnp.concatenate([col, pad_dst]).reshape(NW, NCH, CH)

    zeros1d = jnp.zeros((ROWS_PER_TILE,), jnp.float32)
    zeros2d = jnp.zeros((ROWS_PER_TILE, D), jnp.float32)

    hist = _deg_kernel(col_p, zeros1d)
    h0 = hist[0][:, None]
    h1 = hist[1][:, None]

    x_pad = jnp.concatenate(
        [x, jnp.zeros((NPAD - N, D), jnp.float32)], axis=0)

    h1s_a, h1s_b = _tc1(x_pad, W1, h0, h1)
    h1s = jnp.stack([h1s_a, h1s_b])
    agg1 = _spmm_kernel(row_p, col_p, h1s, zeros2d)
    h2s_a, h2s_b = _tc2(agg1[0], agg1[1], h1s_a, h0, h1, W2, b1.reshape(1, D))
    h2s = jnp.stack([h2s_a, h2s_b])
    agg2 = _spmm_kernel(row_p, col_p, h2s, zeros2d)
    z = _tc3(agg2[0], agg2[1], h2s_a, h0, h1, b2.reshape(1, D))
    return z[:N]
